# baseline (speedup 1.0000x reference)
import functools

import jax
import jax.numpy as jnp
from jax.experimental import pallas as pl
from jax.experimental.pallas import tpu as pltpu

B = 16
POSE_DIM = 72
SD_DIM = 72
FC_DIM = 1024
LATENT_DIM = 256
NUM_EMB = 1024
COMMIT = 0.25


def _conv2d(x, w, b, stride, pad):
    y = jax.lax.conv_general_dilated(
        x, w, (stride, stride), [(pad, pad), (pad, pad)],
        dimension_numbers=("NCHW", "OIHW", "NCHW"))
    return y + b[None, :, None, None]


def _maxpool3x3s2(x):
    return jax.lax.reduce_window(x, -jnp.inf, jax.lax.max, (1, 1, 3, 3), (1, 1, 2, 2), "VALID")


def _alexnet_features(x, p):
    x = jax.nn.relu(_conv2d(x, p["c1w"], p["c1b"], 4, 2))
    x = _maxpool3x3s2(x)
    x = jax.nn.relu(_conv2d(x, p["c2w"], p["c2b"], 1, 2))
    x = _maxpool3x3s2(x)
    x = jax.nn.relu(_conv2d(x, p["c3w"], p["c3b"], 1, 1))
    x = jax.nn.relu(_conv2d(x, p["c4w"], p["c4b"], 1, 1))
    x = jax.nn.relu(_conv2d(x, p["c5w"], p["c5b"], 1, 1))
    x = _maxpool3x3s2(x)
    return x.reshape(x.shape[0], -1)


def _vq_kernel(lat_ref, emb_ref, loss_ref, q_ref, perp_ref):
    lat = lat_ref[...]
    emb = emb_ref[...]
    d = (jnp.sum(lat * lat, axis=1, keepdims=True)
         + jnp.sum(emb * emb, axis=1)[None, :]
         - 2.0 * jax.lax.dot_general(
             lat, emb, (((1,), (1,)), ((), ())),
             preferred_element_type=jnp.float32))
    idx = jnp.argmin(d, axis=1)
    enc = (jax.lax.broadcasted_iota(jnp.int32, (B, NUM_EMB), 1)
           == idx[:, None]).astype(jnp.float32)
    q = jax.lax.dot_general(enc, emb, (((1,), (0,)), ((), ())),
                            preferred_element_type=jnp.float32)
    q_ref[...] = q
    diff = q - lat
    loss_ref[...] = (COMMIT * jnp.mean(diff * diff)).reshape(1, 1)
    avg = jnp.mean(enc, axis=0)
    perp_ref[...] = jnp.exp(-jnp.sum(avg * jnp.log(avg + 1e-10))).reshape(1, 1)


@functools.partial(jax.jit, static_argnames=())
def _vq(latent, emb):
    loss, q, perp = pl.pallas_call(
        _vq_kernel,
        out_shape=(
            jax.ShapeDtypeStruct((1, 1), jnp.float32),
            jax.ShapeDtypeStruct((B, LATENT_DIM), jnp.float32),
            jax.ShapeDtypeStruct((1, 1), jnp.float32),
        ),
    )(latent, emb)
    return loss[0, 0], q, perp[0, 0]


def _condition_encoder(pose, img, img_crop, img_zoom, p):
    pf = jax.nn.relu(pose @ p["ce_fc1w"].T + p["ce_fc1b"])
    imgs = jnp.concatenate([img, img_crop, img_zoom], axis=0)
    f = _alexnet_features(imgs, p)
    f = jax.nn.relu(f @ p["fc6w"].T + p["fc6b"])
    f = jax.nn.relu(f @ p["fc7w"].T + p["fc7b"])
    f1, f2, f3 = jnp.split(f, 3, axis=0)
    h = jnp.concatenate([pf, f1, f2, f3], axis=1)
    return jax.nn.relu(h @ p["ce_fc2w"].T + p["ce_fc2b"])


def kernel(x, pose, img, img_crop, img_zoom, params):
    p = params
    h = jax.nn.relu(x @ p["e_fc1w"].T + p["e_fc1b"])
    h = jax.nn.relu(h @ p["e_fc2w"].T + p["e_fc2b"])
    c = _condition_encoder(pose, img, img_crop, img_zoom, p)
    latent = jnp.concatenate([h, c], axis=1) @ p["e_flw"].T + p["e_flb"]
    loss, q, perp = _vq(latent, p["emb"])
    d = jax.nn.relu(q @ p["d_fc1w"].T + p["d_fc1b"])
    d = jax.nn.relu(d @ p["d_fc2w"].T + p["d_fc2b"])
    c2 = jax.nn.relu(c @ p["d_fc3w"].T + p["d_fc3b"])
    d = jnp.concatenate([d, c2], axis=1)
    d = jax.nn.relu(d @ p["d_fc4w"].T + p["d_fc4b"])
    d = jax.nn.relu(d @ p["d_fc5w"].T + p["d_fc5b"])
    x_recon = d @ p["d_fc6w"].T + p["d_fc6b"]
    return loss, x_recon, perp


# speedup vs baseline: 6.4641x; 6.4641x over previous
import functools

import jax
import jax.numpy as jnp
from jax.experimental import pallas as pl
from jax.experimental.pallas import tpu as pltpu

B = 16
POSE_DIM = 72
SD_DIM = 72
FC_DIM = 1024
LATENT_DIM = 256
NUM_EMB = 1024
COMMIT = 0.25


def _conv2d(x, w, b, stride, pad):
    y = jax.lax.conv_general_dilated(
        x, w, (stride, stride), [(pad, pad), (pad, pad)],
        dimension_numbers=("NCHW", "OIHW", "NCHW"))
    return y + b[None, :, None, None]


def _maxpool3x3s2(x):
    return jax.lax.reduce_window(x, -jnp.inf, jax.lax.max, (1, 1, 3, 3), (1, 1, 2, 2), "VALID")


def _alexnet_features(x, p):
    x = jax.nn.relu(_conv2d(x, p["c1w"], p["c1b"], 4, 2))
    x = _maxpool3x3s2(x)
    x = jax.nn.relu(_conv2d(x, p["c2w"], p["c2b"], 1, 2))
    x = _maxpool3x3s2(x)
    x = jax.nn.relu(_conv2d(x, p["c3w"], p["c3b"], 1, 1))
    x = jax.nn.relu(_conv2d(x, p["c4w"], p["c4b"], 1, 1))
    x = jax.nn.relu(_conv2d(x, p["c5w"], p["c5b"], 1, 1))
    x = _maxpool3x3s2(x)
    return x.reshape(x.shape[0], -1)


def _vq_kernel(lat_ref, emb_ref, loss_ref, q_ref, perp_ref):
    lat = lat_ref[...]
    emb = emb_ref[...]
    d = (jnp.sum(lat * lat, axis=1, keepdims=True)
         + jnp.sum(emb * emb, axis=1)[None, :]
         - 2.0 * jax.lax.dot_general(
             lat, emb, (((1,), (1,)), ((), ())),
             preferred_element_type=jnp.float32))
    idx = jnp.argmin(d, axis=1)
    enc = (jax.lax.broadcasted_iota(jnp.int32, (B, NUM_EMB), 1)
           == idx[:, None]).astype(jnp.float32)
    q = jax.lax.dot_general(enc, emb, (((1,), (0,)), ((), ())),
                            preferred_element_type=jnp.float32)
    q_ref[...] = q
    diff = q - lat
    loss_ref[...] = (COMMIT * jnp.mean(diff * diff)).reshape(1, 1)
    avg = jnp.mean(enc, axis=0)
    perp_ref[...] = jnp.exp(-jnp.sum(avg * jnp.log(avg + 1e-10))).reshape(1, 1)


@functools.partial(jax.jit, static_argnames=())
def _vq(latent, emb):
    loss, q, perp = pl.pallas_call(
        _vq_kernel,
        out_shape=(
            jax.ShapeDtypeStruct((1, 1), jnp.float32),
            jax.ShapeDtypeStruct((B, LATENT_DIM), jnp.float32),
            jax.ShapeDtypeStruct((1, 1), jnp.float32),
        ),
    )(latent, emb)
    return loss[0, 0], q, perp[0, 0]


def _condition_encoder(pose, img, img_crop, img_zoom, p):
    pf = jax.nn.relu(pose @ p["ce_fc1w"].T + p["ce_fc1b"])
    f = jnp.concatenate([_alexnet_features(img, p),
                         _alexnet_features(img_crop, p),
                         _alexnet_features(img_zoom, p)], axis=0)
    f = jax.nn.relu(f @ p["fc6w"].T + p["fc6b"])
    f = jax.nn.relu(f @ p["fc7w"].T + p["fc7b"])
    f1, f2, f3 = jnp.split(f, 3, axis=0)
    h = jnp.concatenate([pf, f1, f2, f3], axis=1)
    return jax.nn.relu(h @ p["ce_fc2w"].T + p["ce_fc2b"])


def kernel(x, pose, img, img_crop, img_zoom, params):
    p = params
    h = jax.nn.relu(x @ p["e_fc1w"].T + p["e_fc1b"])
    h = jax.nn.relu(h @ p["e_fc2w"].T + p["e_fc2b"])
    c = _condition_encoder(pose, img, img_crop, img_zoom, p)
    latent = jnp.concatenate([h, c], axis=1) @ p["e_flw"].T + p["e_flb"]
    loss, q, perp = _vq(latent, p["emb"])
    d = jax.nn.relu(q @ p["d_fc1w"].T + p["d_fc1b"])
    d = jax.nn.relu(d @ p["d_fc2w"].T + p["d_fc2b"])
    c2 = jax.nn.relu(c @ p["d_fc3w"].T + p["d_fc3b"])
    d = jnp.concatenate([d, c2], axis=1)
    d = jax.nn.relu(d @ p["d_fc4w"].T + p["d_fc4b"])
    d = jax.nn.relu(d @ p["d_fc5w"].T + p["d_fc5b"])
    x_recon = d @ p["d_fc6w"].T + p["d_fc6b"]
    return loss, x_recon, perp


# speedup vs baseline: 6.4902x; 1.0040x over previous
import functools

import jax
import jax.numpy as jnp
from jax.experimental import pallas as pl
from jax.experimental.pallas import tpu as pltpu

B = 16
POSE_DIM = 72
SD_DIM = 72
FC_DIM = 1024
LATENT_DIM = 256
NUM_EMB = 1024
COMMIT = 0.25


def _conv2d(x, w, b, stride, pad):
    y = jax.lax.conv_general_dilated(
        x.astype(jnp.bfloat16), w.astype(jnp.bfloat16), (stride, stride),
        [(pad, pad), (pad, pad)],
        dimension_numbers=("NCHW", "OIHW", "NCHW"),
        preferred_element_type=jnp.float32)
    return y + b[None, :, None, None]


def _maxpool3x3s2(x):
    return jax.lax.reduce_window(x, -jnp.inf, jax.lax.max, (1, 1, 3, 3), (1, 1, 2, 2), "VALID")


def _alexnet_features(x, p):
    x = jax.nn.relu(_conv2d(x, p["c1w"], p["c1b"], 4, 2))
    x = _maxpool3x3s2(x)
    x = jax.nn.relu(_conv2d(x, p["c2w"], p["c2b"], 1, 2))
    x = _maxpool3x3s2(x)
    x = jax.nn.relu(_conv2d(x, p["c3w"], p["c3b"], 1, 1))
    x = jax.nn.relu(_conv2d(x, p["c4w"], p["c4b"], 1, 1))
    x = jax.nn.relu(_conv2d(x, p["c5w"], p["c5b"], 1, 1))
    x = _maxpool3x3s2(x)
    return x.reshape(x.shape[0], -1)


def _vq_kernel(lat_ref, emb_ref, loss_ref, q_ref, perp_ref):
    lat = lat_ref[...]
    emb = emb_ref[...]
    d = (jnp.sum(lat * lat, axis=1, keepdims=True)
         + jnp.sum(emb * emb, axis=1)[None, :]
         - 2.0 * jax.lax.dot_general(
             lat, emb, (((1,), (1,)), ((), ())),
             preferred_element_type=jnp.float32))
    idx = jnp.argmin(d, axis=1)
    enc = (jax.lax.broadcasted_iota(jnp.int32, (B, NUM_EMB), 1)
           == idx[:, None]).astype(jnp.float32)
    q = jax.lax.dot_general(enc, emb, (((1,), (0,)), ((), ())),
                            preferred_element_type=jnp.float32)
    q_ref[...] = q
    diff = q - lat
    loss_ref[...] = (COMMIT * jnp.mean(diff * diff)).reshape(1, 1)
    avg = jnp.mean(enc, axis=0)
    perp_ref[...] = jnp.exp(-jnp.sum(avg * jnp.log(avg + 1e-10))).reshape(1, 1)


@functools.partial(jax.jit, static_argnames=())
def _vq(latent, emb):
    loss, q, perp = pl.pallas_call(
        _vq_kernel,
        out_shape=(
            jax.ShapeDtypeStruct((1, 1), jnp.float32),
            jax.ShapeDtypeStruct((B, LATENT_DIM), jnp.float32),
            jax.ShapeDtypeStruct((1, 1), jnp.float32),
        ),
    )(latent, emb)
    return loss[0, 0], q, perp[0, 0]


def _condition_encoder(pose, img, img_crop, img_zoom, p):
    pf = jax.nn.relu(pose @ p["ce_fc1w"].T + p["ce_fc1b"])
    f = jnp.concatenate([_alexnet_features(img, p),
                         _alexnet_features(img_crop, p),
                         _alexnet_features(img_zoom, p)], axis=0)
    f = jax.nn.relu(f @ p["fc6w"].T + p["fc6b"])
    f = jax.nn.relu(f @ p["fc7w"].T + p["fc7b"])
    f1, f2, f3 = jnp.split(f, 3, axis=0)
    h = jnp.concatenate([pf, f1, f2, f3], axis=1)
    return jax.nn.relu(h @ p["ce_fc2w"].T + p["ce_fc2b"])


def kernel(x, pose, img, img_crop, img_zoom, params):
    p = params
    h = jax.nn.relu(x @ p["e_fc1w"].T + p["e_fc1b"])
    h = jax.nn.relu(h @ p["e_fc2w"].T + p["e_fc2b"])
    c = _condition_encoder(pose, img, img_crop, img_zoom, p)
    latent = jnp.concatenate([h, c], axis=1) @ p["e_flw"].T + p["e_flb"]
    loss, q, perp = _vq(latent, p["emb"])
    d = jax.nn.relu(q @ p["d_fc1w"].T + p["d_fc1b"])
    d = jax.nn.relu(d @ p["d_fc2w"].T + p["d_fc2b"])
    c2 = jax.nn.relu(c @ p["d_fc3w"].T + p["d_fc3b"])
    d = jnp.concatenate([d, c2], axis=1)
    d = jax.nn.relu(d @ p["d_fc4w"].T + p["d_fc4b"])
    d = jax.nn.relu(d @ p["d_fc5w"].T + p["d_fc5b"])
    x_recon = d @ p["d_fc6w"].T + p["d_fc6b"]
    return loss, x_recon, perp
